# Initial kernel scaffold; baseline (speedup 1.0000x reference)
import jax
import jax.numpy as jnp
from jax.experimental import pallas as pl


def kernel(xs, x_proj_weight, dt_projs_weight, dt_projs_bias, A_logs, Ds):
    raise NotImplementedError("write your pallas kernel here")


# speedup vs baseline: 1.0811x; 1.0811x over previous
import functools

import jax
import jax.numpy as jnp
from jax.experimental import pallas as pl
from jax.experimental.pallas import tpu as pltpu


def _scan_body(x_ref, wx_ref, wdt_ref, A_ref, d_ref, bias_ref,
               out_ref,
               u_scr, dy_scr, p_scr, dA_scr, dBu_scr, hall_scr,
               *, sub, n_state, dt_rank):
    N = n_state
    R = dt_rank
    D = A_ref.shape[1]
    L = x_ref.shape[1]
    f32 = jnp.float32

    xT = x_ref[...].astype(f32).T
    u_scr[...] = xT

    p_scr[...] = jnp.dot(xT, wx_ref[...], preferred_element_type=f32)
    dts = jnp.dot(p_scr[:, :R], wdt_ref[...],
                  preferred_element_type=f32)
    dy_scr[...] = jax.nn.softplus(dts + bias_ref[...])

    A_b = A_ref[...][None, :, :]
    d_row = d_ref[...]

    col = jax.lax.broadcasted_iota(jnp.int32, (sub, sub * N), 1)
    row = jax.lax.broadcasted_iota(jnp.int32, (sub, sub * N), 0)
    blkmask = (col // N) == row

    n_blk = L // sub

    def blk_body(blk, h):
        base = pl.multiple_of(blk * sub, sub)
        rows = pl.ds(base, sub)
        dl = dy_scr[rows, :]
        uu = u_scr[rows, :]
        bb = p_scr[rows, R:R + N]
        cc = p_scr[rows, R + N:R + 2 * N]

        dA_scr[...] = jnp.exp(dl[:, None, :] * A_b)
        dBu_scr[...] = (dl * uu)[:, None, :] * bb[:, :, None]

        for j in range(sub):
            h = dA_scr[j] * h + dBu_scr[j]
            hall_scr[j] = h

        cc_rep = jnp.tile(cc, (1, sub))
        cblk = jnp.where(blkmask, cc_rep, 0.0)
        hmat = hall_scr[...].reshape(sub * N, D)
        y = jax.lax.dot_general(cblk, hmat, (((1,), (0,)), ((), ())),
                                preferred_element_type=f32)
        dy_scr[rows, :] = y + d_row * uu
        return h

    jax.lax.fori_loop(0, n_blk, blk_body, jnp.zeros((N, D), f32))

    out_ref[...] = dy_scr[...].T.astype(out_ref.dtype)


def _mamba_scan(xs, x_proj_weight, dt_projs_weight, dt_projs_bias,
                A_logs, Ds, *, sub=16):
    B, K, D, L = xs.shape
    N = A_logs.shape[1]
    R = dt_projs_weight.shape[2]
    C = R + 2 * N

    assert L % sub == 0
    f32 = jnp.float32

    wx = jnp.transpose(x_proj_weight, (0, 2, 1)).astype(f32)
    wdt = jnp.transpose(dt_projs_weight, (0, 2, 1)).astype(f32)
    A_nd = jnp.transpose(
        -jnp.exp(A_logs.astype(f32)).reshape(K, D, N), (0, 2, 1))
    d_param = Ds.astype(f32).reshape(K, 1, D)
    bias = dt_projs_bias.astype(f32).reshape(K, 1, D)

    body = functools.partial(_scan_body, sub=sub, n_state=N, dt_rank=R)

    out = pl.pallas_call(
        body,
        out_shape=jax.ShapeDtypeStruct((B, K, D, L), f32),
        grid=(K, B),
        in_specs=[
            pl.BlockSpec((None, None, D, L), lambda k, b: (b, k, 0, 0)),
            pl.BlockSpec((None, D, C), lambda k, b: (k, 0, 0)),
            pl.BlockSpec((None, R, D), lambda k, b: (k, 0, 0)),
            pl.BlockSpec((None, N, D), lambda k, b: (k, 0, 0)),
            pl.BlockSpec((None, 1, D), lambda k, b: (k, 0, 0)),
            pl.BlockSpec((None, 1, D), lambda k, b: (k, 0, 0)),
        ],
        out_specs=pl.BlockSpec((None, None, D, L), lambda k, b: (b, k, 0, 0)),
        scratch_shapes=[
            pltpu.VMEM((L, D), f32),
            pltpu.VMEM((L, D), f32),
            pltpu.VMEM((L, C), f32),
            pltpu.VMEM((sub, N, D), f32),
            pltpu.VMEM((sub, N, D), f32),
            pltpu.VMEM((sub, N, D), f32),
        ],
        compiler_params=pltpu.CompilerParams(
            dimension_semantics=("parallel", "parallel"),
            vmem_limit_bytes=int(40 << 20)),
    )(xs, wx, wdt, A_nd, d_param, bias)

    return out


def kernel(xs, x_proj_weight, dt_projs_weight, dt_projs_bias, A_logs, Ds):
    return _mamba_scan(xs, x_proj_weight, dt_projs_weight, dt_projs_bias,
                       A_logs, Ds, sub=16)


# speedup vs baseline: 1.4216x; 1.3150x over previous
import functools

import jax
import jax.numpy as jnp
from jax.experimental import pallas as pl
from jax.experimental.pallas import tpu as pltpu


def _scan_body(x_ref, wx_ref, wdt_ref, A_ref, d_ref, bias_ref,
               out_ref,
               u_scr, dl_scr, w_scr, y_scr, p_scr,
               dA_a, dA_b, dBu_a, dBu_b, hall_a, hall_b,
               *, sub, n_state, dt_rank):
    N = n_state
    R = dt_rank
    D = A_ref.shape[1]
    L = x_ref.shape[1]
    f32 = jnp.float32
    LOG2E = 1.4426950408889634

    xT = x_ref[...].astype(f32).T
    u_scr[...] = xT

    p_scr[...] = jnp.dot(xT, wx_ref[...], preferred_element_type=f32)
    dts = jnp.dot(p_scr[:, :R], wdt_ref[...],
                  preferred_element_type=f32)
    delta = jax.nn.softplus(dts + bias_ref[...])
    dl_scr[...] = delta * LOG2E
    w_scr[...] = delta * xT

    A_b = A_ref[...][None, :, :]

    col = jax.lax.broadcasted_iota(jnp.int32, (sub, sub * N), 1)
    row = jax.lax.broadcasted_iota(jnp.int32, (sub, sub * N), 0)
    blkmask = (col // N) == row

    def half_block(base, h, dA, dBu, hall):
        rows = pl.ds(base, sub)
        dl = dl_scr[rows, :]
        w = w_scr[rows, :]
        bb = p_scr[rows, R:R + N]
        cc = p_scr[rows, R + N:R + 2 * N]

        dA[...] = jnp.exp2(dl[:, None, :] * A_b)
        dBu[...] = w[:, None, :] * bb[:, :, None]

        for j in range(sub):
            h = dA[j] * h + dBu[j]
            hall[j] = h

        cc_rep = jnp.tile(cc, (1, sub))
        cblk = jnp.where(blkmask, cc_rep, 0.0)
        hmat = hall[...].reshape(sub * N, D)
        y = jax.lax.dot_general(cblk, hmat, (((1,), (0,)), ((), ())),
                                preferred_element_type=f32)
        y_scr[rows, :] = y
        return h

    n_pair = L // (2 * sub)

    def pair_body(p, h):
        base = pl.multiple_of(p * (2 * sub), 2 * sub)
        h = half_block(base, h, dA_a, dBu_a, hall_a)
        h = half_block(base + sub, h, dA_b, dBu_b, hall_b)
        return h

    jax.lax.fori_loop(0, n_pair, pair_body, jnp.zeros((N, D), f32))

    yT = y_scr[...] + d_ref[...] * u_scr[...]
    out_ref[...] = yT.T.astype(out_ref.dtype)


def _mamba_scan(xs, x_proj_weight, dt_projs_weight, dt_projs_bias,
                A_logs, Ds, *, sub=16):
    B, K, D, L = xs.shape
    N = A_logs.shape[1]
    R = dt_projs_weight.shape[2]
    C = R + 2 * N

    assert L % (2 * sub) == 0
    f32 = jnp.float32

    wx = jnp.transpose(x_proj_weight, (0, 2, 1)).astype(f32)
    wdt = jnp.transpose(dt_projs_weight, (0, 2, 1)).astype(f32)
    A_nd = jnp.transpose(
        -jnp.exp(A_logs.astype(f32)).reshape(K, D, N), (0, 2, 1))
    d_param = Ds.astype(f32).reshape(K, 1, D)
    bias = dt_projs_bias.astype(f32).reshape(K, 1, D)

    body = functools.partial(_scan_body, sub=sub, n_state=N, dt_rank=R)

    out = pl.pallas_call(
        body,
        out_shape=jax.ShapeDtypeStruct((B, K, D, L), f32),
        grid=(K, B),
        in_specs=[
            pl.BlockSpec((None, None, D, L), lambda k, b: (b, k, 0, 0)),
            pl.BlockSpec((None, D, C), lambda k, b: (k, 0, 0)),
            pl.BlockSpec((None, R, D), lambda k, b: (k, 0, 0)),
            pl.BlockSpec((None, N, D), lambda k, b: (k, 0, 0)),
            pl.BlockSpec((None, 1, D), lambda k, b: (k, 0, 0)),
            pl.BlockSpec((None, 1, D), lambda k, b: (k, 0, 0)),
        ],
        out_specs=pl.BlockSpec((None, None, D, L), lambda k, b: (b, k, 0, 0)),
        scratch_shapes=[
            pltpu.VMEM((L, D), f32),
            pltpu.VMEM((L, D), f32),
            pltpu.VMEM((L, D), f32),
            pltpu.VMEM((L, D), f32),
            pltpu.VMEM((L, C), f32),
            pltpu.VMEM((sub, N, D), f32),
            pltpu.VMEM((sub, N, D), f32),
            pltpu.VMEM((sub, N, D), f32),
            pltpu.VMEM((sub, N, D), f32),
            pltpu.VMEM((sub, N, D), f32),
            pltpu.VMEM((sub, N, D), f32),
        ],
        compiler_params=pltpu.CompilerParams(
            dimension_semantics=("parallel", "parallel"),
            vmem_limit_bytes=int(40 << 20)),
    )(xs, wx, wdt, A_nd, d_param, bias)

    return out


def kernel(xs, x_proj_weight, dt_projs_weight, dt_projs_bias, A_logs, Ds):
    return _mamba_scan(xs, x_proj_weight, dt_projs_weight, dt_projs_bias,
                       A_logs, Ds, sub=16)


# speedup vs baseline: 1.5241x; 1.0721x over previous
import functools

import jax
import jax.numpy as jnp
from jax.experimental import pallas as pl
from jax.experimental.pallas import tpu as pltpu


def _scan_body(x_ref, wx_ref, wdt_ref, A_ref, d_ref, bias_ref,
               out_ref,
               u_scr, dl_scr, w_scr, y_scr, p_scr,
               dA_a, dA_b, dBu_a, dBu_b, hall_a, hall_b,
               *, sub, n_state, dt_rank):
    N = n_state
    R = dt_rank
    D = A_ref.shape[1]
    L = x_ref.shape[1]
    f32 = jnp.float32
    LOG2E = 1.4426950408889634

    xT = x_ref[...].astype(f32).T
    u_scr[...] = xT

    p_scr[...] = jnp.dot(xT, wx_ref[...], preferred_element_type=f32)
    dts = jnp.dot(p_scr[:, :R], wdt_ref[...],
                  preferred_element_type=f32)
    delta = jax.nn.softplus(dts + bias_ref[...])
    dl_scr[...] = (delta * LOG2E)[:, None, :]
    w_scr[...] = (delta * xT)[:, None, :]

    A_b = A_ref[...][None, :, :]

    col = jax.lax.broadcasted_iota(jnp.int32, (sub, sub * N), 1)
    row = jax.lax.broadcasted_iota(jnp.int32, (sub, sub * N), 0)
    blkmask = (col // N) == row

    def half_block(base, h, dA, dBu, hall):
        rows = pl.ds(base, sub)
        dl = dl_scr[rows]
        w = w_scr[rows]
        bb = p_scr[rows, R:R + N]
        cc = p_scr[rows, R + N:R + 2 * N]

        dA[...] = jnp.exp2(dl * A_b)
        dBu[...] = w * bb[:, :, None]

        for j in range(sub):
            h = dA[j] * h + dBu[j]
            hall[j] = h

        cc_rep = jnp.tile(cc, (1, sub))
        cblk = jnp.where(blkmask, cc_rep, 0.0)
        hmat = hall[...].reshape(sub * N, D)
        y = jax.lax.dot_general(cblk, hmat, (((1,), (0,)), ((), ())),
                                preferred_element_type=f32)
        y_scr[rows, :] = y
        return h

    n_pair = L // (2 * sub)

    def pair_body(p, h):
        base = pl.multiple_of(p * (2 * sub), 2 * sub)
        h = half_block(base, h, dA_a, dBu_a, hall_a)
        h = half_block(base + sub, h, dA_b, dBu_b, hall_b)
        return h

    jax.lax.fori_loop(0, n_pair, pair_body, jnp.zeros((N, D), f32))

    yT = y_scr[...] + d_ref[...] * u_scr[...]
    out_ref[...] = yT.T.astype(out_ref.dtype)


def _mamba_scan(xs, x_proj_weight, dt_projs_weight, dt_projs_bias,
                A_logs, Ds, *, sub=16):
    B, K, D, L = xs.shape
    N = A_logs.shape[1]
    R = dt_projs_weight.shape[2]
    C = R + 2 * N

    assert L % (2 * sub) == 0
    f32 = jnp.float32

    wx = jnp.transpose(x_proj_weight, (0, 2, 1)).astype(f32)
    wdt = jnp.transpose(dt_projs_weight, (0, 2, 1)).astype(f32)
    A_nd = jnp.transpose(
        -jnp.exp(A_logs.astype(f32)).reshape(K, D, N), (0, 2, 1))
    d_param = Ds.astype(f32).reshape(K, 1, D)
    bias = dt_projs_bias.astype(f32).reshape(K, 1, D)

    body = functools.partial(_scan_body, sub=sub, n_state=N, dt_rank=R)

    out = pl.pallas_call(
        body,
        out_shape=jax.ShapeDtypeStruct((B, K, D, L), f32),
        grid=(K, B),
        in_specs=[
            pl.BlockSpec((None, None, D, L), lambda k, b: (b, k, 0, 0)),
            pl.BlockSpec((None, D, C), lambda k, b: (k, 0, 0)),
            pl.BlockSpec((None, R, D), lambda k, b: (k, 0, 0)),
            pl.BlockSpec((None, N, D), lambda k, b: (k, 0, 0)),
            pl.BlockSpec((None, 1, D), lambda k, b: (k, 0, 0)),
            pl.BlockSpec((None, 1, D), lambda k, b: (k, 0, 0)),
        ],
        out_specs=pl.BlockSpec((None, None, D, L), lambda k, b: (b, k, 0, 0)),
        scratch_shapes=[
            pltpu.VMEM((L, D), f32),
            pltpu.VMEM((L, 1, D), f32),
            pltpu.VMEM((L, 1, D), f32),
            pltpu.VMEM((L, D), f32),
            pltpu.VMEM((L, C), f32),
            pltpu.VMEM((sub, N, D), f32),
            pltpu.VMEM((sub, N, D), f32),
            pltpu.VMEM((sub, N, D), f32),
            pltpu.VMEM((sub, N, D), f32),
            pltpu.VMEM((sub, N, D), f32),
            pltpu.VMEM((sub, N, D), f32),
        ],
        compiler_params=pltpu.CompilerParams(
            dimension_semantics=("parallel", "parallel"),
            vmem_limit_bytes=int(40 << 20)),
    )(xs, wx, wdt, A_nd, d_param, bias)

    return out


def kernel(xs, x_proj_weight, dt_projs_weight, dt_projs_bias, A_logs, Ds):
    return _mamba_scan(xs, x_proj_weight, dt_projs_weight, dt_projs_bias,
                       A_logs, Ds, sub=16)


# speedup vs baseline: 1.7863x; 1.1720x over previous
import functools

import jax
import jax.numpy as jnp
from jax.experimental import pallas as pl
from jax.experimental.pallas import tpu as pltpu


def _scan_body(x_ref, wx_ref, wdt_ref, A_ref, d_ref, bias_ref,
               out_ref,
               u_scr, dl_scr, w_scr, y_scr, p_scr,
               dA_a, dA_b, dBu_a, dBu_b, hall_scr,
               *, sub, n_state, dt_rank):
    N = n_state
    R = dt_rank
    D = A_ref.shape[1]
    L = x_ref.shape[1]
    f32 = jnp.float32
    LOG2E = 1.4426950408889634

    xT = x_ref[...].astype(f32).T
    u_scr[...] = xT

    p_scr[...] = jnp.dot(xT, wx_ref[...], preferred_element_type=f32)
    dts = jnp.dot(p_scr[:, :R], wdt_ref[...],
                  preferred_element_type=f32)
    delta = jax.nn.softplus(dts + bias_ref[...])
    dl_scr[...] = (delta * LOG2E)[:, None, :]
    w_scr[...] = (delta * xT)[:, None, :]

    A_b = A_ref[...][None, :, :]

    col = jax.lax.broadcasted_iota(jnp.int32, (sub, sub * N), 1)
    row = jax.lax.broadcasted_iota(jnp.int32, (sub, sub * N), 0)
    blkmask = (col // N) == row

    def construct(base, dA, dBu):
        rows = pl.ds(base, sub)
        dA[...] = jnp.exp2(dl_scr[rows] * A_b)
        dBu[...] = w_scr[rows] * p_scr[rows, R:R + N][:, :, None]

    def chain(base, h, dA, dBu):
        for j in range(sub):
            h = dA[j] * h + dBu[j]
            hall_scr[base + j] = h
        return h

    n_pair = L // (2 * sub)

    def pair_body(p, h):
        base = pl.multiple_of(p * (2 * sub), 2 * sub)
        construct(base, dA_a, dBu_a)
        construct(base + sub, dA_b, dBu_b)
        h = chain(base, h, dA_a, dBu_a)
        h = chain(base + sub, h, dA_b, dBu_b)
        return h

    jax.lax.fori_loop(0, n_pair, pair_body, jnp.zeros((N, D), f32))

    for blk in range(L // sub):
        base = blk * sub
        cc = p_scr[base:base + sub, R + N:R + 2 * N]
        cc_rep = jnp.tile(cc, (1, sub))
        cblk = jnp.where(blkmask, cc_rep, 0.0)
        hmat = hall_scr[base:base + sub].reshape(sub * N, D)
        y = jax.lax.dot_general(cblk, hmat, (((1,), (0,)), ((), ())),
                                preferred_element_type=f32)
        y_scr[base:base + sub, :] = y

    yT = y_scr[...] + d_ref[...] * u_scr[...]
    out_ref[...] = yT.T.astype(out_ref.dtype)


def _mamba_scan(xs, x_proj_weight, dt_projs_weight, dt_projs_bias,
                A_logs, Ds, *, sub=16):
    B, K, D, L = xs.shape
    N = A_logs.shape[1]
    R = dt_projs_weight.shape[2]
    C = R + 2 * N

    assert L % (2 * sub) == 0
    f32 = jnp.float32

    wx = jnp.transpose(x_proj_weight, (0, 2, 1)).astype(f32)
    wdt = jnp.transpose(dt_projs_weight, (0, 2, 1)).astype(f32)
    A_nd = jnp.transpose(
        -jnp.exp(A_logs.astype(f32)).reshape(K, D, N), (0, 2, 1))
    d_param = Ds.astype(f32).reshape(K, 1, D)
    bias = dt_projs_bias.astype(f32).reshape(K, 1, D)

    body = functools.partial(_scan_body, sub=sub, n_state=N, dt_rank=R)

    out = pl.pallas_call(
        body,
        out_shape=jax.ShapeDtypeStruct((B, K, D, L), f32),
        grid=(K, B),
        in_specs=[
            pl.BlockSpec((None, None, D, L), lambda k, b: (b, k, 0, 0)),
            pl.BlockSpec((None, D, C), lambda k, b: (k, 0, 0)),
            pl.BlockSpec((None, R, D), lambda k, b: (k, 0, 0)),
            pl.BlockSpec((None, N, D), lambda k, b: (k, 0, 0)),
            pl.BlockSpec((None, 1, D), lambda k, b: (k, 0, 0)),
            pl.BlockSpec((None, 1, D), lambda k, b: (k, 0, 0)),
        ],
        out_specs=pl.BlockSpec((None, None, D, L), lambda k, b: (b, k, 0, 0)),
        scratch_shapes=[
            pltpu.VMEM((L, D), f32),
            pltpu.VMEM((L, 1, D), f32),
            pltpu.VMEM((L, 1, D), f32),
            pltpu.VMEM((L, D), f32),
            pltpu.VMEM((L, C), f32),
            pltpu.VMEM((sub, N, D), f32),
            pltpu.VMEM((sub, N, D), f32),
            pltpu.VMEM((sub, N, D), f32),
            pltpu.VMEM((sub, N, D), f32),
            pltpu.VMEM((L, N, D), f32),
        ],
        compiler_params=pltpu.CompilerParams(
            dimension_semantics=("parallel", "parallel"),
            vmem_limit_bytes=int(56 << 20)),
    )(xs, wx, wdt, A_nd, d_param, bias)

    return out


def kernel(xs, x_proj_weight, dt_projs_weight, dt_projs_bias, A_logs, Ds):
    return _mamba_scan(xs, x_proj_weight, dt_projs_weight, dt_projs_bias,
                       A_logs, Ds, sub=16)


# speedup vs baseline: 1.8094x; 1.0129x over previous
import functools

import jax
import jax.numpy as jnp
from jax.experimental import pallas as pl
from jax.experimental.pallas import tpu as pltpu


def _scan_body(x_ref, wx_ref, wdt_ref, A_ref, d_ref, bias_ref,
               out_ref,
               dl_scr, w_scr, y_scr, p_scr,
               dA_a, dA_b, dBu_a, dBu_b, hall_scr,
               *, sub, n_state, dt_rank):
    N = n_state
    R = dt_rank
    D = A_ref.shape[1]
    L = x_ref.shape[1]
    f32 = jnp.float32
    LOG2E = 1.4426950408889634

    xT = x_ref[...].astype(f32).T

    p_scr[...] = jnp.dot(xT, wx_ref[...], preferred_element_type=f32)
    dts = jnp.dot(p_scr[:, :R], wdt_ref[...],
                  preferred_element_type=f32)
    delta = jax.nn.softplus(dts + bias_ref[...])
    dl_scr[...] = (delta * LOG2E)[:, None, :]
    w_scr[...] = (delta * xT)[:, None, :]

    A_b = A_ref[...][None, :, :]

    col = jax.lax.broadcasted_iota(jnp.int32, (sub, sub * N), 1)
    row = jax.lax.broadcasted_iota(jnp.int32, (sub, sub * N), 0)
    blkmask = (col // N) == row

    def construct(base, dA, dBu):
        rows = pl.ds(base, sub)
        dA[...] = jnp.exp2(dl_scr[rows] * A_b)
        dBu[...] = w_scr[rows] * p_scr[rows, R:R + N][:, :, None]

    def chain(base, h, dA, dBu):
        for j in range(sub):
            h = dA[j] * h + dBu[j]
            hall_scr[base + j] = h
        return h

    n_pair = L // (2 * sub)

    def pair_body(p, h):
        base = pl.multiple_of(p * (2 * sub), 2 * sub)
        construct(base, dA_a, dBu_a)
        construct(base + sub, dA_b, dBu_b)
        h = chain(base, h, dA_a, dBu_a)
        h = chain(base + sub, h, dA_b, dBu_b)
        return h

    jax.lax.fori_loop(0, n_pair, pair_body, jnp.zeros((N, D), f32))

    for blk in range(L // sub):
        base = blk * sub
        cc = p_scr[base:base + sub, R + N:R + 2 * N]
        cc_rep = jnp.tile(cc, (1, sub))
        cblk = jnp.where(blkmask, cc_rep, 0.0)
        hmat = hall_scr[base:base + sub].reshape(sub * N, D)
        y = jax.lax.dot_general(cblk, hmat, (((1,), (0,)), ((), ())),
                                preferred_element_type=f32)
        y_scr[base:base + sub, :] = y

    out_ref[...] = (y_scr[...].T + d_ref[...] * x_ref[...]).astype(out_ref.dtype)


def _mamba_scan(xs, x_proj_weight, dt_projs_weight, dt_projs_bias,
                A_logs, Ds, *, sub=16):
    B, K, D, L = xs.shape
    N = A_logs.shape[1]
    R = dt_projs_weight.shape[2]
    C = R + 2 * N

    assert L % (2 * sub) == 0
    f32 = jnp.float32

    wx = jnp.transpose(x_proj_weight, (0, 2, 1)).astype(f32)
    wdt = jnp.transpose(dt_projs_weight, (0, 2, 1)).astype(f32)
    A_nd = jnp.transpose(
        -jnp.exp(A_logs.astype(f32)).reshape(K, D, N), (0, 2, 1))
    d_param = Ds.astype(f32).reshape(K, D, 1)
    bias = dt_projs_bias.astype(f32).reshape(K, 1, D)

    body = functools.partial(_scan_body, sub=sub, n_state=N, dt_rank=R)

    out = pl.pallas_call(
        body,
        out_shape=jax.ShapeDtypeStruct((B, K, D, L), f32),
        grid=(K, B),
        in_specs=[
            pl.BlockSpec((None, None, D, L), lambda k, b: (b, k, 0, 0)),
            pl.BlockSpec((None, D, C), lambda k, b: (k, 0, 0)),
            pl.BlockSpec((None, R, D), lambda k, b: (k, 0, 0)),
            pl.BlockSpec((None, N, D), lambda k, b: (k, 0, 0)),
            pl.BlockSpec((None, D, 1), lambda k, b: (k, 0, 0)),
            pl.BlockSpec((None, 1, D), lambda k, b: (k, 0, 0)),
        ],
        out_specs=pl.BlockSpec((None, None, D, L), lambda k, b: (b, k, 0, 0)),
        scratch_shapes=[
            pltpu.VMEM((L, 1, D), f32),
            pltpu.VMEM((L, 1, D), f32),
            pltpu.VMEM((L, D), f32),
            pltpu.VMEM((L, C), f32),
            pltpu.VMEM((sub, N, D), f32),
            pltpu.VMEM((sub, N, D), f32),
            pltpu.VMEM((sub, N, D), f32),
            pltpu.VMEM((sub, N, D), f32),
            pltpu.VMEM((L, N, D), f32),
        ],
        compiler_params=pltpu.CompilerParams(
            dimension_semantics=("parallel", "parallel"),
            vmem_limit_bytes=int(56 << 20)),
    )(xs, wx, wdt, A_nd, d_param, bias)

    return out


def kernel(xs, x_proj_weight, dt_projs_weight, dt_projs_bias, A_logs, Ds):
    return _mamba_scan(xs, x_proj_weight, dt_projs_weight, dt_projs_bias,
                       A_logs, Ds, sub=32)


# speedup vs baseline: 1.8418x; 1.0179x over previous
import functools

import jax
import jax.numpy as jnp
from jax.experimental import pallas as pl
from jax.experimental.pallas import tpu as pltpu


def _scan_body(x_ref, wx_ref, wdt_ref, A_ref, d_ref, bias_ref,
               out_ref,
               dl_scr, w_scr, y_scr, p_scr,
               dA_a, dA_b, dBu_a, dBu_b, hall_scr,
               *, sub, n_state, dt_rank):
    N = n_state
    R = dt_rank
    D = A_ref.shape[1]
    L = x_ref.shape[1]
    f32 = jnp.float32
    LOG2E = 1.4426950408889634

    xT = x_ref[...].astype(f32).T

    p_scr[...] = jnp.dot(xT, wx_ref[...], preferred_element_type=f32)
    dts = jnp.dot(p_scr[:, :R], wdt_ref[...],
                  preferred_element_type=f32)
    delta = jax.nn.softplus(dts + bias_ref[...])
    dl_scr[...] = (delta * LOG2E)[:, None, :]
    w_scr[...] = (delta * xT)[:, None, :]

    A_b = A_ref[...][None, :, :]

    col = jax.lax.broadcasted_iota(jnp.int32, (sub, sub * N), 1)
    row = jax.lax.broadcasted_iota(jnp.int32, (sub, sub * N), 0)
    blkmask = (col // N) == row

    def construct(base, dA, dBu):
        rows = pl.ds(base, sub)
        dA[...] = jnp.exp2(dl_scr[rows] * A_b)
        dBu[...] = w_scr[rows] * p_scr[rows, R:R + N][:, :, None]

    def chain(base, h, dA, dBu):
        for j in range(sub):
            h = dA[j] * h + dBu[j]
            hall_scr[base + j] = h
        return h

    n_pair = L // (2 * sub)

    def pair_body(p, h):
        base = pl.multiple_of(p * (2 * sub), 2 * sub)
        construct(base, dA_a, dBu_a)
        construct(base + sub, dA_b, dBu_b)
        h = chain(base, h, dA_a, dBu_a)
        h = chain(base + sub, h, dA_b, dBu_b)
        return h

    jax.lax.fori_loop(0, n_pair, pair_body, jnp.zeros((N, D), f32))

    for blk in range(L // sub):
        base = blk * sub
        cc = p_scr[base:base + sub, R + N:R + 2 * N]
        cc_rep = jnp.tile(cc, (1, sub))
        cblk = jnp.where(blkmask, cc_rep, 0.0)
        hmat = hall_scr[base:base + sub].reshape(sub * N, D)
        y = jax.lax.dot_general(cblk, hmat, (((1,), (0,)), ((), ())),
                                preferred_element_type=f32)
        y_scr[base:base + sub, :] = y

    out_ref[...] = (y_scr[...].T + d_ref[...] * x_ref[...]).astype(out_ref.dtype)


def _mamba_scan(xs, x_proj_weight, dt_projs_weight, dt_projs_bias,
                A_logs, Ds, *, sub=16):
    B, K, D, L = xs.shape
    N = A_logs.shape[1]
    R = dt_projs_weight.shape[2]
    C = R + 2 * N

    assert L % (2 * sub) == 0
    f32 = jnp.float32

    wx = jnp.transpose(x_proj_weight, (0, 2, 1)).astype(f32)
    wdt = jnp.transpose(dt_projs_weight, (0, 2, 1)).astype(f32)
    A_nd = jnp.transpose(
        -jnp.exp(A_logs.astype(f32)).reshape(K, D, N), (0, 2, 1))
    d_param = Ds.astype(f32).reshape(K, D, 1)
    bias = dt_projs_bias.astype(f32).reshape(K, 1, D)

    body = functools.partial(_scan_body, sub=sub, n_state=N, dt_rank=R)

    out = pl.pallas_call(
        body,
        out_shape=jax.ShapeDtypeStruct((B, K, D, L), f32),
        grid=(K, B),
        in_specs=[
            pl.BlockSpec((None, None, D, L), lambda k, b: (b, k, 0, 0)),
            pl.BlockSpec((None, D, C), lambda k, b: (k, 0, 0)),
            pl.BlockSpec((None, R, D), lambda k, b: (k, 0, 0)),
            pl.BlockSpec((None, N, D), lambda k, b: (k, 0, 0)),
            pl.BlockSpec((None, D, 1), lambda k, b: (k, 0, 0)),
            pl.BlockSpec((None, 1, D), lambda k, b: (k, 0, 0)),
        ],
        out_specs=pl.BlockSpec((None, None, D, L), lambda k, b: (b, k, 0, 0)),
        scratch_shapes=[
            pltpu.VMEM((L, 1, D), f32),
            pltpu.VMEM((L, 1, D), f32),
            pltpu.VMEM((L, D), f32),
            pltpu.VMEM((L, C), f32),
            pltpu.VMEM((sub, N, D), f32),
            pltpu.VMEM((sub, N, D), f32),
            pltpu.VMEM((sub, N, D), f32),
            pltpu.VMEM((sub, N, D), f32),
            pltpu.VMEM((L, N, D), f32),
        ],
        compiler_params=pltpu.CompilerParams(
            dimension_semantics=("parallel", "parallel"),
            vmem_limit_bytes=int(56 << 20)),
    )(xs, wx, wdt, A_nd, d_param, bias)

    return out


def kernel(xs, x_proj_weight, dt_projs_weight, dt_projs_bias, A_logs, Ds):
    return _mamba_scan(xs, x_proj_weight, dt_projs_weight, dt_projs_bias,
                       A_logs, Ds, sub=64)
